# baseline (speedup 1.0000x reference)
import functools

import jax
import jax.numpy as jnp
from jax import lax
from jax.experimental import pallas as pl
from jax.experimental.pallas import tpu as pltpu
from jax.experimental.pallas import tpu_sc as plsc

N = 100000
D = 128
S = 1024
EPS = 1e-5

NC = 2
NS = 16
NW = NC * NS
RPW = 3120
TAIL = NW * RPW
TCH = 80
LPR = D // 16

ACH = 80
ACPW = RPW // ACH

UNIT = 104
UPW = RPW // UNIT
SLOTS = 4
XLEAD = 2
WIN = 112

_mesh = plsc.VectorSubcoreMesh(core_axis_name="c", subcore_axis_name="s")


@functools.partial(
    pl.kernel,
    out_type=jax.ShapeDtypeStruct((NC * S, D), jnp.float32),
    mesh=_mesh,
    scratch_types=[
        pltpu.VMEM((ACH, D), jnp.float32),
        pltpu.VMEM((ACH, D), jnp.float32),
        pltpu.VMEM((RPW,), jnp.int32),
        pltpu.VMEM((ACH,), jnp.int32),
        pltpu.VMEM_SHARED((S, D), jnp.float32),
        pltpu.SemaphoreType.DMA,
        pltpu.SemaphoreType.DMA,
    ],
)
def _segsum_k(x_hbm, b_hbm, out_hbm, bufx0, bufx1, idxs, idx80, acc,
              semx0, semx1):
    cid = lax.axis_index("c")
    sid = lax.axis_index("s")
    wid = sid * NC + cid
    r0 = pl.multiple_of(wid * RPW, 8)
    rows_per_tile = S // NS
    bufx = (bufx0, bufx1)
    semx = (semx0, semx1)

    z = jnp.zeros((16,), jnp.float32)

    def zbody(r, _):
        for k in range(LPR):
            bufx0[r, pl.ds(k * 16, 16)] = z
        return 0

    lax.fori_loop(0, rows_per_tile, zbody, 0)
    pltpu.sync_copy(bufx0.at[pl.ds(0, rows_per_tile)],
                    acc.at[pl.ds(sid * rows_per_tile, rows_per_tile)])

    pltpu.sync_copy(b_hbm.at[pl.ds(r0, RPW)], idxs)
    plsc.subcore_barrier()

    def x_load(i, b):
        return pltpu.async_copy(
            x_hbm.at[pl.ds(r0 + i * ACH, ACH)], bufx[b], semx[b])

    d = x_load(0, 0)
    for i in range(ACPW):
        b = i % 2
        d.wait()
        if i + 1 < ACPW:
            d = x_load(i + 1, 1 - b)
        for k in range(ACH // 16):
            idx80[pl.ds(k * 16, 16)] = idxs[pl.ds(i * ACH + k * 16, 16)]
        pltpu.sync_copy(bufx[b], acc.at[idx80], add=True)

    @pl.when(wid < 2)
    def _():
        t0 = pl.multiple_of(TAIL + wid * TCH, 8)
        pltpu.sync_copy(x_hbm.at[pl.ds(t0, TCH)], bufx0)
        pltpu.sync_copy(b_hbm.at[pl.ds(t0, TCH)], idx80)
        pltpu.sync_copy(bufx0, acc.at[idx80], add=True)

    plsc.subcore_barrier()
    pltpu.sync_copy(
        acc.at[pl.ds(sid * rows_per_tile, rows_per_tile)],
        out_hbm.at[pl.ds(cid * S + sid * rows_per_tile, rows_per_tile)])


def _mlp_body(hp_ref, w1_ref, b1_ref, g_ref, be_ref, w2_ref, b2_ref, vn_ref):
    h = hp_ref[0] + hp_ref[1]
    z = lax.dot_general(h, w1_ref[...], (((1,), (1,)), ((), ())),
                        preferred_element_type=jnp.float32) + b1_ref[...]
    mu = jnp.mean(z, axis=0, keepdims=True)
    var = jnp.mean(jnp.square(z - mu), axis=0, keepdims=True)
    zn = (z - mu) * lax.rsqrt(var + EPS) * g_ref[...] + be_ref[...]
    a = jnp.maximum(zn, 0.0)
    vn = lax.dot_general(a, w2_ref[...], (((1,), (1,)), ((), ())),
                         preferred_element_type=jnp.float32) + b2_ref[...]
    vn_ref[...] = vn


_mlp = pl.pallas_call(
    _mlp_body,
    out_shape=jax.ShapeDtypeStruct((S, D), jnp.float32),
)


@functools.partial(
    pl.kernel,
    out_type=jax.ShapeDtypeStruct((N, D), jnp.float32),
    mesh=_mesh,
    scratch_types=(
        [pltpu.VMEM((UNIT, D), jnp.float32) for _ in range(SLOTS)]
        + [pltpu.VMEM((WIN, D), jnp.float32) for _ in range(SLOTS)]
        + [pltpu.VMEM((RPW + 16,), jnp.int32),
           pltpu.VMEM((TCH,), jnp.int32)]
        + [pltpu.SemaphoreType.DMA for _ in range(3 * SLOTS)]
    ),
)
def _bcast_k(x_hbm, b_hbm, vn_hbm, out_hbm, *refs):
    bufs = refs[:SLOTS]
    wins = refs[SLOTS:2 * SLOTS]
    idxs = refs[2 * SLOTS]
    ixt = refs[2 * SLOTS + 1]
    semx = refs[2 * SLOTS + 2:3 * SLOTS + 2]
    semw = refs[3 * SLOTS + 2:4 * SLOTS + 2]
    semo = refs[4 * SLOTS + 2:5 * SLOTS + 2]
    cid = lax.axis_index("c")
    sid = lax.axis_index("s")
    wid = sid * NC + cid
    r0 = pl.multiple_of(wid * RPW, 8)

    pltpu.sync_copy(b_hbm.at[pl.ds(r0, RPW)], idxs.at[pl.ds(0, RPW)])

    def rows(u):
        return pl.ds(r0 + u * UNIT, UNIT)

    def win_base(u):
        f = idxs[pl.ds(u * UNIT, 16)][0]
        return jnp.minimum((f >> 3) << 3, S - WIN)

    def loads(u):
        k = u % SLOTS
        dx = pltpu.async_copy(x_hbm.at[rows(u)], bufs[k], semx[k])
        base = pl.multiple_of(win_base(u), 8)
        dw = pltpu.async_copy(vn_hbm.at[pl.ds(base, WIN)], wins[k], semw[k])
        return (dx, dw)

    def add_unit(u):
        k = u % SLOTS
        base = win_base(u)
        last = idxs[pl.ds(u * UNIT + UNIT - 16, 16)][15]
        in_win = (last - base) < WIN

        @pl.when(in_win)
        def _():
            def row_body(r, _):
                off = idxs[pl.ds(u * UNIT + r, 16)][0] - base
                for q in range(LPR):
                    bufs[k][r, pl.ds(q * 16, 16)] = (
                        bufs[k][r, pl.ds(q * 16, 16)]
                        + wins[k][off, pl.ds(q * 16, 16)])
                return 0

            lax.fori_loop(0, UNIT, row_body, 0)

        @pl.when(jnp.logical_not(in_win))
        def _():
            pltpu.sync_copy(vn_hbm.at[idxs.at[pl.ds(u * UNIT, UNIT)]],
                            bufs[k], add=True)

    def store(u):
        k = u % SLOTS
        return pltpu.async_copy(bufs[k], out_hbm.at[rows(u)], semo[k])

    dl, do = {}, {}
    for u in range(min(XLEAD, UPW)):
        dl[u % SLOTS] = loads(u)
    for u in range(UPW):
        k = u % SLOTS
        if u + XLEAD < UPW:
            kn = (u + XLEAD) % SLOTS
            if u >= SLOTS - XLEAD:
                do.pop(kn).wait()
            dl[kn] = loads(u + XLEAD)
        for dd in dl.pop(k):
            dd.wait()
        add_unit(u)
        do[k] = store(u)
    for dd in do.values():
        dd.wait()

    @pl.when(wid < 2)
    def _():
        t0 = pl.multiple_of(TAIL + wid * TCH, 8)
        pltpu.sync_copy(x_hbm.at[pl.ds(t0, TCH)], bufs[0].at[pl.ds(0, TCH)])
        pltpu.sync_copy(b_hbm.at[pl.ds(t0, TCH)], ixt)
        pltpu.sync_copy(vn_hbm.at[ixt], bufs[0].at[pl.ds(0, TCH)], add=True)
        pltpu.sync_copy(bufs[0].at[pl.ds(0, TCH)], out_hbm.at[pl.ds(t0, TCH)])


def kernel(x, batch, W1, b1, gamma, beta, W2, b2):
    batch32 = batch.astype(jnp.int32)
    hp = _segsum_k(x, batch32).reshape(NC, S, D)
    vn = _mlp(hp, W1, b1.reshape(1, D), gamma.reshape(1, D),
              beta.reshape(1, D), W2, b2.reshape(1, D))
    return _bcast_k(x, batch32, vn)


# speedup vs baseline: 2.2452x; 2.2452x over previous
import functools

import jax
import jax.numpy as jnp
from jax import lax
from jax.experimental import pallas as pl
from jax.experimental.pallas import tpu as pltpu
from jax.experimental.pallas import tpu_sc as plsc

N = 100000
D = 128
S = 1024
EPS = 1e-5

NC = 2
NS = 16
NW = NC * NS
RPW = 3120
TAIL = NW * RPW
TCH = 80
LPR = D // 16

ACH = 80
ACPW = RPW // ACH

UNIT = 104
UPW = RPW // UNIT
SLOTS = 7
XLEAD = 3
GLAG = 3

_mesh = plsc.VectorSubcoreMesh(core_axis_name="c", subcore_axis_name="s")


@functools.partial(
    pl.kernel,
    out_type=jax.ShapeDtypeStruct((NC * S, D), jnp.float32),
    mesh=_mesh,
    scratch_types=[
        pltpu.VMEM((ACH, D), jnp.float32),
        pltpu.VMEM((ACH, D), jnp.float32),
        pltpu.VMEM((RPW,), jnp.int32),
        pltpu.VMEM((ACH,), jnp.int32),
        pltpu.VMEM_SHARED((S, D), jnp.float32),
        pltpu.SemaphoreType.DMA,
        pltpu.SemaphoreType.DMA,
    ],
)
def _segsum_k(x_hbm, b_hbm, out_hbm, bufx0, bufx1, idxs, idx80, acc,
              semx0, semx1):
    cid = lax.axis_index("c")
    sid = lax.axis_index("s")
    wid = sid * NC + cid
    r0 = pl.multiple_of(wid * RPW, 8)
    rows_per_tile = S // NS
    bufx = (bufx0, bufx1)
    semx = (semx0, semx1)

    z = jnp.zeros((16,), jnp.float32)

    def zbody(r, _):
        for k in range(LPR):
            bufx0[r, pl.ds(k * 16, 16)] = z
        return 0

    lax.fori_loop(0, rows_per_tile, zbody, 0)
    pltpu.sync_copy(bufx0.at[pl.ds(0, rows_per_tile)],
                    acc.at[pl.ds(sid * rows_per_tile, rows_per_tile)])

    pltpu.sync_copy(b_hbm.at[pl.ds(r0, RPW)], idxs)
    plsc.subcore_barrier()

    def x_load(i, b):
        return pltpu.async_copy(
            x_hbm.at[pl.ds(r0 + i * ACH, ACH)], bufx[b], semx[b])

    d = x_load(0, 0)
    for i in range(ACPW):
        b = i % 2
        d.wait()
        if i + 1 < ACPW:
            d = x_load(i + 1, 1 - b)
        for k in range(ACH // 16):
            idx80[pl.ds(k * 16, 16)] = idxs[pl.ds(i * ACH + k * 16, 16)]
        pltpu.sync_copy(bufx[b], acc.at[idx80], add=True)

    @pl.when(wid < 2)
    def _():
        t0 = pl.multiple_of(TAIL + wid * TCH, 8)
        pltpu.sync_copy(x_hbm.at[pl.ds(t0, TCH)], bufx0)
        pltpu.sync_copy(b_hbm.at[pl.ds(t0, TCH)], idx80)
        pltpu.sync_copy(bufx0, acc.at[idx80], add=True)

    plsc.subcore_barrier()
    pltpu.sync_copy(
        acc.at[pl.ds(sid * rows_per_tile, rows_per_tile)],
        out_hbm.at[pl.ds(cid * S + sid * rows_per_tile, rows_per_tile)])


def _mlp_body(hp_ref, w1_ref, b1_ref, g_ref, be_ref, w2_ref, b2_ref, vn_ref):
    h = hp_ref[0] + hp_ref[1]
    z = lax.dot_general(h, w1_ref[...], (((1,), (1,)), ((), ())),
                        preferred_element_type=jnp.float32) + b1_ref[...]
    mu = jnp.mean(z, axis=0, keepdims=True)
    var = jnp.mean(jnp.square(z - mu), axis=0, keepdims=True)
    zn = (z - mu) * lax.rsqrt(var + EPS) * g_ref[...] + be_ref[...]
    a = jnp.maximum(zn, 0.0)
    vn = lax.dot_general(a, w2_ref[...], (((1,), (1,)), ((), ())),
                         preferred_element_type=jnp.float32) + b2_ref[...]
    vn_ref[...] = vn


_mlp = pl.pallas_call(
    _mlp_body,
    out_shape=jax.ShapeDtypeStruct((S, D), jnp.float32),
)


@functools.partial(
    pl.kernel,
    out_type=jax.ShapeDtypeStruct((N, D), jnp.float32),
    mesh=_mesh,
    scratch_types=(
        [pltpu.VMEM((UNIT, D), jnp.float32) for _ in range(SLOTS)]
        + [pltpu.VMEM((RPW,), jnp.int32),
           pltpu.VMEM((TCH,), jnp.int32),
           pltpu.VMEM_SHARED((S, D), jnp.float32)]
        + [pltpu.SemaphoreType.DMA for _ in range(3 * SLOTS)]
    ),
)
def _bcast_k(x_hbm, b_hbm, vn_hbm, out_hbm, *refs):
    bufs = refs[:SLOTS]
    idxs = refs[SLOTS]
    ixt = refs[SLOTS + 1]
    vnsp = refs[SLOTS + 2]
    semx = refs[SLOTS + 3:2 * SLOTS + 3]
    semv = refs[2 * SLOTS + 3:3 * SLOTS + 3]
    semo = refs[3 * SLOTS + 3:4 * SLOTS + 3]
    cid = lax.axis_index("c")
    sid = lax.axis_index("s")
    wid = sid * NC + cid
    r0 = pl.multiple_of(wid * RPW, 8)
    rows_per_tile = S // NS

    pltpu.sync_copy(vn_hbm.at[pl.ds(sid * rows_per_tile, rows_per_tile)],
                    vnsp.at[pl.ds(sid * rows_per_tile, rows_per_tile)])
    pltpu.sync_copy(b_hbm.at[pl.ds(r0, RPW)], idxs)
    plsc.subcore_barrier()

    def rows(u):
        return pl.ds(r0 + u * UNIT, UNIT)

    def x_load(u):
        k = u % SLOTS
        return pltpu.async_copy(x_hbm.at[rows(u)], bufs[k], semx[k])

    def v_gather_add(u):
        k = u % SLOTS
        return pltpu.async_copy(
            vnsp.at[idxs.at[pl.ds(u * UNIT, UNIT)]], bufs[k], semv[k],
            add=True)

    def store(u):
        k = u % SLOTS
        return pltpu.async_copy(bufs[k], out_hbm.at[rows(u)], semo[k])

    dx, dv, do = {}, {}, {}
    for u in range(min(XLEAD, UPW)):
        dx[u % SLOTS] = x_load(u)
    for u in range(UPW):
        k = u % SLOTS
        if u + XLEAD < UPW:
            kn = (u + XLEAD) % SLOTS
            if u >= SLOTS - XLEAD:
                do.pop(kn).wait()
            dx[kn] = x_load(u + XLEAD)
        dx.pop(k).wait()
        dv[k] = v_gather_add(u)
        if u >= GLAG:
            kr = (u - GLAG) % SLOTS
            dv.pop(kr).wait()
            do[kr] = store(u - GLAG)
    for u in range(max(0, UPW - GLAG), UPW):
        kr = u % SLOTS
        dv.pop(kr).wait()
        do[kr] = store(u)
    for dd in do.values():
        dd.wait()

    @pl.when(wid < 2)
    def _():
        t0 = pl.multiple_of(TAIL + wid * TCH, 8)
        pltpu.sync_copy(x_hbm.at[pl.ds(t0, TCH)], bufs[0].at[pl.ds(0, TCH)])
        pltpu.sync_copy(b_hbm.at[pl.ds(t0, TCH)], ixt)
        pltpu.sync_copy(vnsp.at[ixt], bufs[0].at[pl.ds(0, TCH)], add=True)
        pltpu.sync_copy(bufs[0].at[pl.ds(0, TCH)], out_hbm.at[pl.ds(t0, TCH)])


def kernel(x, batch, W1, b1, gamma, beta, W2, b2):
    batch32 = batch.astype(jnp.int32)
    hp = _segsum_k(x, batch32).reshape(NC, S, D)
    vn = _mlp(hp, W1, b1.reshape(1, D), gamma.reshape(1, D),
              beta.reshape(1, D), W2, b2.reshape(1, D))
    return _bcast_k(x, batch32, vn)


# speedup vs baseline: 2.4686x; 1.0995x over previous
import functools

import jax
import jax.numpy as jnp
from jax import lax
from jax.experimental import pallas as pl
from jax.experimental.pallas import tpu as pltpu
from jax.experimental.pallas import tpu_sc as plsc

N = 100000
D = 128
S = 1024
EPS = 1e-5

NC = 2
NS = 16
NW = NC * NS
RPW = 3120
TAIL = NW * RPW
TCH = 80
LPR = D // 16

ACH = 80
ACPW = RPW // ACH

UNIT = 104
UPW = RPW // UNIT
SLOTS = 7
XLEAD = 3
GLAG = 3

_mesh = plsc.VectorSubcoreMesh(core_axis_name="c", subcore_axis_name="s")


@functools.partial(
    pl.kernel,
    out_type=jax.ShapeDtypeStruct((NC * S, D), jnp.float32),
    mesh=_mesh,
    scratch_types=(
        [pltpu.VMEM((ACH, D), jnp.float32) for _ in range(4)]
        + [pltpu.VMEM((ACH,), jnp.int32) for _ in range(4)]
        + [pltpu.VMEM((RPW,), jnp.int32),
           pltpu.VMEM_SHARED((S, D), jnp.float32)]
        + [pltpu.SemaphoreType.DMA for _ in range(8)]
    ),
)
def _segsum_k(x_hbm, b_hbm, out_hbm, *refs):
    bufx = refs[0:4]
    ix = refs[4:8]
    idxs = refs[8]
    acc = refs[9]
    semx = refs[10:14]
    sems = refs[14:18]
    cid = lax.axis_index("c")
    sid = lax.axis_index("s")
    wid = sid * NC + cid
    r0 = pl.multiple_of(wid * RPW, 8)
    rows_per_tile = S // NS

    z = jnp.zeros((16,), jnp.float32)

    def zbody(r, _):
        for k in range(LPR):
            bufx[0][r, pl.ds(k * 16, 16)] = z
        return 0

    lax.fori_loop(0, rows_per_tile, zbody, 0)
    pltpu.sync_copy(bufx[0].at[pl.ds(0, rows_per_tile)],
                    acc.at[pl.ds(sid * rows_per_tile, rows_per_tile)])

    pltpu.sync_copy(b_hbm.at[pl.ds(r0, RPW)], idxs)
    plsc.subcore_barrier()

    def x_load(i):
        k = i % 4
        return pltpu.async_copy(
            x_hbm.at[pl.ds(r0 + i * ACH, ACH)], bufx[k], semx[k])

    dx = {0: x_load(0), 1: x_load(1)}
    ds = {}
    for i in range(ACPW):
        k = i % 4
        dx.pop(k).wait()
        for q in range(ACH // 16):
            ix[k][pl.ds(q * 16, 16)] = idxs[pl.ds(i * ACH + q * 16, 16)]
        ds[k] = pltpu.async_copy(bufx[k], acc.at[ix[k]], sems[k], add=True)
        if i + 2 < ACPW:
            kn = (i + 2) % 4
            if i >= 2:
                ds.pop(kn).wait()
            dx[kn] = x_load(i + 2)
    for dd in ds.values():
        dd.wait()

    @pl.when(wid < 2)
    def _():
        t0 = pl.multiple_of(TAIL + wid * TCH, 8)
        pltpu.sync_copy(x_hbm.at[pl.ds(t0, TCH)], bufx[0])
        pltpu.sync_copy(b_hbm.at[pl.ds(t0, TCH)], ix[0])
        pltpu.sync_copy(bufx[0], acc.at[ix[0]], add=True)

    plsc.subcore_barrier()
    pltpu.sync_copy(
        acc.at[pl.ds(sid * rows_per_tile, rows_per_tile)],
        out_hbm.at[pl.ds(cid * S + sid * rows_per_tile, rows_per_tile)])


def _mlp_body(hp_ref, w1_ref, b1_ref, g_ref, be_ref, w2_ref, b2_ref, vn_ref):
    h = hp_ref[0] + hp_ref[1]
    z = lax.dot_general(h, w1_ref[...], (((1,), (1,)), ((), ())),
                        preferred_element_type=jnp.float32) + b1_ref[...]
    mu = jnp.mean(z, axis=0, keepdims=True)
    var = jnp.mean(jnp.square(z - mu), axis=0, keepdims=True)
    zn = (z - mu) * lax.rsqrt(var + EPS) * g_ref[...] + be_ref[...]
    a = jnp.maximum(zn, 0.0)
    vn = lax.dot_general(a, w2_ref[...], (((1,), (1,)), ((), ())),
                         preferred_element_type=jnp.float32) + b2_ref[...]
    vn_ref[...] = vn


_mlp = pl.pallas_call(
    _mlp_body,
    out_shape=jax.ShapeDtypeStruct((S, D), jnp.float32),
)


@functools.partial(
    pl.kernel,
    out_type=jax.ShapeDtypeStruct((N, D), jnp.float32),
    mesh=_mesh,
    scratch_types=(
        [pltpu.VMEM((UNIT, D), jnp.float32) for _ in range(SLOTS)]
        + [pltpu.VMEM((RPW,), jnp.int32),
           pltpu.VMEM((TCH,), jnp.int32),
           pltpu.VMEM_SHARED((S, D), jnp.float32)]
        + [pltpu.SemaphoreType.DMA for _ in range(3 * SLOTS)]
    ),
)
def _bcast_k(x_hbm, b_hbm, vn_hbm, out_hbm, *refs):
    bufs = refs[:SLOTS]
    idxs = refs[SLOTS]
    ixt = refs[SLOTS + 1]
    vnsp = refs[SLOTS + 2]
    semx = refs[SLOTS + 3:2 * SLOTS + 3]
    semv = refs[2 * SLOTS + 3:3 * SLOTS + 3]
    semo = refs[3 * SLOTS + 3:4 * SLOTS + 3]
    cid = lax.axis_index("c")
    sid = lax.axis_index("s")
    wid = sid * NC + cid
    r0 = pl.multiple_of(wid * RPW, 8)
    rows_per_tile = S // NS

    pltpu.sync_copy(vn_hbm.at[pl.ds(sid * rows_per_tile, rows_per_tile)],
                    vnsp.at[pl.ds(sid * rows_per_tile, rows_per_tile)])
    pltpu.sync_copy(b_hbm.at[pl.ds(r0, RPW)], idxs)
    plsc.subcore_barrier()

    def rows(u):
        return pl.ds(r0 + u * UNIT, UNIT)

    def x_load(u):
        k = u % SLOTS
        return pltpu.async_copy(x_hbm.at[rows(u)], bufs[k], semx[k])

    def v_gather_add(u):
        k = u % SLOTS
        return pltpu.async_copy(
            vnsp.at[idxs.at[pl.ds(u * UNIT, UNIT)]], bufs[k], semv[k],
            add=True)

    def store(u):
        k = u % SLOTS
        return pltpu.async_copy(bufs[k], out_hbm.at[rows(u)], semo[k])

    dx, dv, do = {}, {}, {}
    for u in range(min(XLEAD, UPW)):
        dx[u % SLOTS] = x_load(u)
    for u in range(UPW):
        k = u % SLOTS
        if u + XLEAD < UPW:
            kn = (u + XLEAD) % SLOTS
            if u >= SLOTS - XLEAD:
                do.pop(kn).wait()
            dx[kn] = x_load(u + XLEAD)
        dx.pop(k).wait()
        dv[k] = v_gather_add(u)
        if u >= GLAG:
            kr = (u - GLAG) % SLOTS
            dv.pop(kr).wait()
            do[kr] = store(u - GLAG)
    for u in range(max(0, UPW - GLAG), UPW):
        kr = u % SLOTS
        dv.pop(kr).wait()
        do[kr] = store(u)
    for dd in do.values():
        dd.wait()

    @pl.when(wid < 2)
    def _():
        t0 = pl.multiple_of(TAIL + wid * TCH, 8)
        pltpu.sync_copy(x_hbm.at[pl.ds(t0, TCH)], bufs[0].at[pl.ds(0, TCH)])
        pltpu.sync_copy(b_hbm.at[pl.ds(t0, TCH)], ixt)
        pltpu.sync_copy(vnsp.at[ixt], bufs[0].at[pl.ds(0, TCH)], add=True)
        pltpu.sync_copy(bufs[0].at[pl.ds(0, TCH)], out_hbm.at[pl.ds(t0, TCH)])


def kernel(x, batch, W1, b1, gamma, beta, W2, b2):
    batch32 = batch.astype(jnp.int32)
    hp = _segsum_k(x, batch32).reshape(NC, S, D)
    vn = _mlp(hp, W1, b1.reshape(1, D), gamma.reshape(1, D),
              beta.reshape(1, D), W2, b2.reshape(1, D))
    return _bcast_k(x, batch32, vn)


# speedup vs baseline: 2.4939x; 1.0102x over previous
import functools

import jax
import jax.numpy as jnp
from jax import lax
from jax.experimental import pallas as pl
from jax.experimental.pallas import tpu as pltpu
from jax.experimental.pallas import tpu_sc as plsc

N = 100000
D = 128
S = 1024
EPS = 1e-5

NC = 2
NS = 16
NW = NC * NS
RPW = 3120
TAIL = NW * RPW
TCH = 80
LPR = D // 16

ACH = 104
ACPW = RPW // ACH
ASL = 6
AXL = 2

UNIT = 104
UPW = RPW // UNIT
SLOTS = 8
XLEAD = 3
GLAG = 4

_mesh = plsc.VectorSubcoreMesh(core_axis_name="c", subcore_axis_name="s")


@functools.partial(
    pl.kernel,
    out_type=jax.ShapeDtypeStruct((NC * S, D), jnp.float32),
    mesh=_mesh,
    scratch_types=(
        [pltpu.VMEM((ACH, D), jnp.float32) for _ in range(ASL)]
        + [pltpu.VMEM((ACH,), jnp.int32) for _ in range(ASL)]
        + [pltpu.VMEM((RPW,), jnp.int32),
           pltpu.VMEM((TCH,), jnp.int32),
           pltpu.VMEM_SHARED((S, D), jnp.float32)]
        + [pltpu.SemaphoreType.DMA for _ in range(2 * ASL)]
    ),
)
def _segsum_k(x_hbm, b_hbm, out_hbm, *refs):
    bufx = refs[:ASL]
    ix = refs[ASL:2 * ASL]
    idxs = refs[2 * ASL]
    ixt = refs[2 * ASL + 1]
    acc = refs[2 * ASL + 2]
    semx = refs[2 * ASL + 3:3 * ASL + 3]
    sems = refs[3 * ASL + 3:4 * ASL + 3]
    cid = lax.axis_index("c")
    sid = lax.axis_index("s")
    wid = sid * NC + cid
    r0 = pl.multiple_of(wid * RPW, 8)
    rows_per_tile = S // NS

    z = jnp.zeros((16,), jnp.float32)

    def zbody(r, _):
        for k in range(LPR):
            bufx[0][r, pl.ds(k * 16, 16)] = z
        return 0

    lax.fori_loop(0, rows_per_tile, zbody, 0)
    pltpu.sync_copy(bufx[0].at[pl.ds(0, rows_per_tile)],
                    acc.at[pl.ds(sid * rows_per_tile, rows_per_tile)])

    pltpu.sync_copy(b_hbm.at[pl.ds(r0, RPW)], idxs)
    plsc.subcore_barrier()

    def x_load(i):
        k = i % ASL
        return pltpu.async_copy(
            x_hbm.at[pl.ds(r0 + i * ACH, ACH)], bufx[k], semx[k])

    _ROFF = [0, 16, 32, 48, 64, 80, ACH - 16]

    dx = {}
    ds = {}
    for u in range(min(AXL, ACPW)):
        dx[u % ASL] = x_load(u)
    for i in range(ACPW):
        k = i % ASL
        if i + AXL < ACPW:
            kn = (i + AXL) % ASL
            if i >= ASL - AXL:
                ds.pop(kn).wait()
            dx[kn] = x_load(i + AXL)
        dx.pop(k).wait()
        for q in _ROFF:
            ix[k][pl.ds(q, 16)] = idxs[pl.ds(i * ACH + q, 16)]
        ds[k] = pltpu.async_copy(bufx[k], acc.at[ix[k]], sems[k], add=True)
    for dd in ds.values():
        dd.wait()

    @pl.when(wid < 2)
    def _():
        t0 = pl.multiple_of(TAIL + wid * TCH, 8)
        pltpu.sync_copy(x_hbm.at[pl.ds(t0, TCH)], bufx[0].at[pl.ds(0, TCH)])
        pltpu.sync_copy(b_hbm.at[pl.ds(t0, TCH)], ixt)
        pltpu.sync_copy(bufx[0].at[pl.ds(0, TCH)], acc.at[ixt], add=True)

    plsc.subcore_barrier()
    pltpu.sync_copy(
        acc.at[pl.ds(sid * rows_per_tile, rows_per_tile)],
        out_hbm.at[pl.ds(cid * S + sid * rows_per_tile, rows_per_tile)])


def _mlp_body(hp_ref, w1_ref, b1_ref, g_ref, be_ref, w2_ref, b2_ref, vn_ref):
    h = hp_ref[0] + hp_ref[1]
    z = lax.dot_general(h, w1_ref[...], (((1,), (1,)), ((), ())),
                        preferred_element_type=jnp.float32) + b1_ref[...]
    mu = jnp.mean(z, axis=0, keepdims=True)
    var = jnp.mean(jnp.square(z - mu), axis=0, keepdims=True)
    zn = (z - mu) * lax.rsqrt(var + EPS) * g_ref[...] + be_ref[...]
    a = jnp.maximum(zn, 0.0)
    vn = lax.dot_general(a, w2_ref[...], (((1,), (1,)), ((), ())),
                         preferred_element_type=jnp.float32) + b2_ref[...]
    vn_ref[...] = vn


_mlp = pl.pallas_call(
    _mlp_body,
    out_shape=jax.ShapeDtypeStruct((S, D), jnp.float32),
)


@functools.partial(
    pl.kernel,
    out_type=jax.ShapeDtypeStruct((N, D), jnp.float32),
    mesh=_mesh,
    scratch_types=(
        [pltpu.VMEM((UNIT, D), jnp.float32) for _ in range(SLOTS)]
        + [pltpu.VMEM((RPW,), jnp.int32),
           pltpu.VMEM((TCH,), jnp.int32),
           pltpu.VMEM_SHARED((S, D), jnp.float32)]
        + [pltpu.SemaphoreType.DMA for _ in range(3 * SLOTS)]
    ),
)
def _bcast_k(x_hbm, b_hbm, vn_hbm, out_hbm, *refs):
    bufs = refs[:SLOTS]
    idxs = refs[SLOTS]
    ixt = refs[SLOTS + 1]
    vnsp = refs[SLOTS + 2]
    semx = refs[SLOTS + 3:2 * SLOTS + 3]
    semv = refs[2 * SLOTS + 3:3 * SLOTS + 3]
    semo = refs[3 * SLOTS + 3:4 * SLOTS + 3]
    cid = lax.axis_index("c")
    sid = lax.axis_index("s")
    wid = sid * NC + cid
    r0 = pl.multiple_of(wid * RPW, 8)
    rows_per_tile = S // NS

    pltpu.sync_copy(vn_hbm.at[pl.ds(sid * rows_per_tile, rows_per_tile)],
                    vnsp.at[pl.ds(sid * rows_per_tile, rows_per_tile)])
    pltpu.sync_copy(b_hbm.at[pl.ds(r0, RPW)], idxs)
    plsc.subcore_barrier()

    def rows(u):
        return pl.ds(r0 + u * UNIT, UNIT)

    def x_load(u):
        k = u % SLOTS
        return pltpu.async_copy(x_hbm.at[rows(u)], bufs[k], semx[k])

    def v_gather_add(u):
        k = u % SLOTS
        return pltpu.async_copy(
            vnsp.at[idxs.at[pl.ds(u * UNIT, UNIT)]], bufs[k], semv[k],
            add=True)

    def store(u):
        k = u % SLOTS
        return pltpu.async_copy(bufs[k], out_hbm.at[rows(u)], semo[k])

    dx, dv, do = {}, {}, {}
    for u in range(min(XLEAD, UPW)):
        dx[u % SLOTS] = x_load(u)
    for u in range(UPW):
        k = u % SLOTS
        if u + XLEAD < UPW:
            kn = (u + XLEAD) % SLOTS
            if u >= SLOTS - XLEAD:
                do.pop(kn).wait()
            dx[kn] = x_load(u + XLEAD)
        dx.pop(k).wait()
        dv[k] = v_gather_add(u)
        if u >= GLAG:
            kr = (u - GLAG) % SLOTS
            dv.pop(kr).wait()
            do[kr] = store(u - GLAG)
    for u in range(max(0, UPW - GLAG), UPW):
        kr = u % SLOTS
        dv.pop(kr).wait()
        do[kr] = store(u)
    for dd in do.values():
        dd.wait()

    @pl.when(wid < 2)
    def _():
        t0 = pl.multiple_of(TAIL + wid * TCH, 8)
        pltpu.sync_copy(x_hbm.at[pl.ds(t0, TCH)], bufs[0].at[pl.ds(0, TCH)])
        pltpu.sync_copy(b_hbm.at[pl.ds(t0, TCH)], ixt)
        pltpu.sync_copy(vnsp.at[ixt], bufs[0].at[pl.ds(0, TCH)], add=True)
        pltpu.sync_copy(bufs[0].at[pl.ds(0, TCH)], out_hbm.at[pl.ds(t0, TCH)])


def kernel(x, batch, W1, b1, gamma, beta, W2, b2):
    batch32 = batch.astype(jnp.int32)
    hp = _segsum_k(x, batch32).reshape(NC, S, D)
    vn = _mlp(hp, W1, b1.reshape(1, D), gamma.reshape(1, D),
              beta.reshape(1, D), W2, b2.reshape(1, D))
    return _bcast_k(x, batch32, vn)


# speedup vs baseline: 2.5131x; 1.0077x over previous
import functools

import jax
import jax.numpy as jnp
from jax import lax
from jax.experimental import pallas as pl
from jax.experimental.pallas import tpu as pltpu
from jax.experimental.pallas import tpu_sc as plsc

N = 100000
D = 128
S = 1024
EPS = 1e-5

NC = 2
NS = 16
NW = NC * NS
RPW = 3120
TAIL = NW * RPW
TCH = 80
LPR = D // 16

ACH = 104
ACPW = RPW // ACH
ASL = 8
AXL = 3

UNIT = 104
UPW = RPW // UNIT
SLOTS = 8
XLEAD = 3
GLAG = 4

_mesh = plsc.VectorSubcoreMesh(core_axis_name="c", subcore_axis_name="s")


@functools.partial(
    pl.kernel,
    out_type=jax.ShapeDtypeStruct((NC * S, D), jnp.float32),
    mesh=_mesh,
    scratch_types=(
        [pltpu.VMEM((ACH, D), jnp.float32) for _ in range(ASL)]
        + [pltpu.VMEM((ACH,), jnp.int32) for _ in range(ASL)]
        + [pltpu.VMEM((RPW,), jnp.int32),
           pltpu.VMEM((TCH,), jnp.int32),
           pltpu.VMEM_SHARED((S, D), jnp.float32)]
        + [pltpu.SemaphoreType.DMA for _ in range(2 * ASL)]
    ),
)
def _segsum_k(x_hbm, b_hbm, out_hbm, *refs):
    bufx = refs[:ASL]
    ix = refs[ASL:2 * ASL]
    idxs = refs[2 * ASL]
    ixt = refs[2 * ASL + 1]
    acc = refs[2 * ASL + 2]
    semx = refs[2 * ASL + 3:3 * ASL + 3]
    sems = refs[3 * ASL + 3:4 * ASL + 3]
    cid = lax.axis_index("c")
    sid = lax.axis_index("s")
    wid = sid * NC + cid
    r0 = pl.multiple_of(wid * RPW, 8)
    rows_per_tile = S // NS

    z = jnp.zeros((16,), jnp.float32)

    def zbody(r, _):
        for k in range(LPR):
            bufx[0][r, pl.ds(k * 16, 16)] = z
        return 0

    lax.fori_loop(0, rows_per_tile, zbody, 0)
    pltpu.sync_copy(bufx[0].at[pl.ds(0, rows_per_tile)],
                    acc.at[pl.ds(sid * rows_per_tile, rows_per_tile)])

    pltpu.sync_copy(b_hbm.at[pl.ds(r0, RPW)], idxs)
    plsc.subcore_barrier()

    def x_load(i):
        k = i % ASL
        return pltpu.async_copy(
            x_hbm.at[pl.ds(r0 + i * ACH, ACH)], bufx[k], semx[k])

    _ROFF = [0, 16, 32, 48, 64, 80, ACH - 16]

    dx = {}
    ds = {}
    for u in range(min(AXL, ACPW)):
        dx[u % ASL] = x_load(u)
    for i in range(ACPW):
        k = i % ASL
        if i + AXL < ACPW:
            kn = (i + AXL) % ASL
            if i >= ASL - AXL:
                ds.pop(kn).wait()
            dx[kn] = x_load(i + AXL)
        dx.pop(k).wait()
        for q in _ROFF:
            ix[k][pl.ds(q, 16)] = idxs[pl.ds(i * ACH + q, 16)]
        ds[k] = pltpu.async_copy(bufx[k], acc.at[ix[k]], sems[k], add=True)
    for dd in ds.values():
        dd.wait()

    @pl.when(wid < 2)
    def _():
        t0 = pl.multiple_of(TAIL + wid * TCH, 8)
        pltpu.sync_copy(x_hbm.at[pl.ds(t0, TCH)], bufx[0].at[pl.ds(0, TCH)])
        pltpu.sync_copy(b_hbm.at[pl.ds(t0, TCH)], ixt)
        pltpu.sync_copy(bufx[0].at[pl.ds(0, TCH)], acc.at[ixt], add=True)

    plsc.subcore_barrier()
    pltpu.sync_copy(
        acc.at[pl.ds(sid * rows_per_tile, rows_per_tile)],
        out_hbm.at[pl.ds(cid * S + sid * rows_per_tile, rows_per_tile)])


def _mlp_body(hp_ref, w1_ref, b1_ref, g_ref, be_ref, w2_ref, b2_ref, vn_ref):
    h = hp_ref[0] + hp_ref[1]
    z = lax.dot_general(h, w1_ref[...], (((1,), (1,)), ((), ())),
                        preferred_element_type=jnp.float32) + b1_ref[...]
    mu = jnp.mean(z, axis=0, keepdims=True)
    var = jnp.mean(jnp.square(z - mu), axis=0, keepdims=True)
    zn = (z - mu) * lax.rsqrt(var + EPS) * g_ref[...] + be_ref[...]
    a = jnp.maximum(zn, 0.0)
    vn = lax.dot_general(a, w2_ref[...], (((1,), (1,)), ((), ())),
                         preferred_element_type=jnp.float32) + b2_ref[...]
    vn_ref[...] = vn


_mlp = pl.pallas_call(
    _mlp_body,
    out_shape=jax.ShapeDtypeStruct((S, D), jnp.float32),
)


@functools.partial(
    pl.kernel,
    out_type=jax.ShapeDtypeStruct((N, D), jnp.float32),
    mesh=_mesh,
    scratch_types=(
        [pltpu.VMEM((UNIT, D), jnp.float32) for _ in range(SLOTS)]
        + [pltpu.VMEM((RPW,), jnp.int32),
           pltpu.VMEM((TCH,), jnp.int32),
           pltpu.VMEM_SHARED((S, D), jnp.float32)]
        + [pltpu.SemaphoreType.DMA for _ in range(3 * SLOTS)]
    ),
)
def _bcast_k(x_hbm, b_hbm, vn_hbm, out_hbm, *refs):
    bufs = refs[:SLOTS]
    idxs = refs[SLOTS]
    ixt = refs[SLOTS + 1]
    vnsp = refs[SLOTS + 2]
    semx = refs[SLOTS + 3:2 * SLOTS + 3]
    semv = refs[2 * SLOTS + 3:3 * SLOTS + 3]
    semo = refs[3 * SLOTS + 3:4 * SLOTS + 3]
    cid = lax.axis_index("c")
    sid = lax.axis_index("s")
    wid = sid * NC + cid
    r0 = pl.multiple_of(wid * RPW, 8)
    rows_per_tile = S // NS

    pltpu.sync_copy(vn_hbm.at[pl.ds(sid * rows_per_tile, rows_per_tile)],
                    vnsp.at[pl.ds(sid * rows_per_tile, rows_per_tile)])
    pltpu.sync_copy(b_hbm.at[pl.ds(r0, RPW)], idxs)
    plsc.subcore_barrier()

    def rows(u):
        return pl.ds(r0 + u * UNIT, UNIT)

    def x_load(u):
        k = u % SLOTS
        return pltpu.async_copy(x_hbm.at[rows(u)], bufs[k], semx[k])

    def v_gather_add(u):
        k = u % SLOTS
        return pltpu.async_copy(
            vnsp.at[idxs.at[pl.ds(u * UNIT, UNIT)]], bufs[k], semv[k],
            add=True)

    def store(u):
        k = u % SLOTS
        return pltpu.async_copy(bufs[k], out_hbm.at[rows(u)], semo[k])

    dx, dv, do = {}, {}, {}
    for u in range(min(XLEAD, UPW)):
        dx[u % SLOTS] = x_load(u)
    for u in range(UPW):
        k = u % SLOTS
        if u + XLEAD < UPW:
            kn = (u + XLEAD) % SLOTS
            if u >= SLOTS - XLEAD:
                do.pop(kn).wait()
            dx[kn] = x_load(u + XLEAD)
        dx.pop(k).wait()
        dv[k] = v_gather_add(u)
        if u >= GLAG:
            kr = (u - GLAG) % SLOTS
            dv.pop(kr).wait()
            do[kr] = store(u - GLAG)
    for u in range(max(0, UPW - GLAG), UPW):
        kr = u % SLOTS
        dv.pop(kr).wait()
        do[kr] = store(u)
    for dd in do.values():
        dd.wait()

    @pl.when(wid < 2)
    def _():
        t0 = pl.multiple_of(TAIL + wid * TCH, 8)
        pltpu.sync_copy(x_hbm.at[pl.ds(t0, TCH)], bufs[0].at[pl.ds(0, TCH)])
        pltpu.sync_copy(b_hbm.at[pl.ds(t0, TCH)], ixt)
        pltpu.sync_copy(vnsp.at[ixt], bufs[0].at[pl.ds(0, TCH)], add=True)
        pltpu.sync_copy(bufs[0].at[pl.ds(0, TCH)], out_hbm.at[pl.ds(t0, TCH)])


def kernel(x, batch, W1, b1, gamma, beta, W2, b2):
    batch32 = batch.astype(jnp.int32)
    hp = _segsum_k(x, batch32).reshape(NC, S, D)
    vn = _mlp(hp, W1, b1.reshape(1, D), gamma.reshape(1, D),
              beta.reshape(1, D), W2, b2.reshape(1, D))
    return _bcast_k(x, batch32, vn)
